# baseline (speedup 1.0000x reference)
import functools

import jax
import jax.numpy as jnp
from jax import lax
from jax.experimental import pallas as pl
from jax.experimental.pallas import tpu as pltpu
from jax.experimental.pallas import tpu_sc as plsc

D_MODEL = 1024
BATCH = 16384
NUM_CORES = 2
NUM_SUBCORES = 16
NUM_WORKERS = NUM_CORES * NUM_SUBCORES
B_PER_W = BATCH // NUM_WORKERS
CHUNK = 64
N_CHUNKS = B_PER_W // CHUNK

_mesh = plsc.VectorSubcoreMesh(core_axis_name="c", subcore_axis_name="s")


@functools.partial(
    pl.kernel,
    mesh=_mesh,
    out_type=jax.ShapeDtypeStruct((BATCH, D_MODEL), jnp.float32),
    scratch_types=[
        pltpu.VMEM((B_PER_W,), jnp.int32),
        pltpu.VMEM((CHUNK, D_MODEL), jnp.float32),
        pltpu.SemaphoreType.DMA,
    ],
)
def _emb_lookup(weeks_hbm, table_hbm, out_hbm, idx_v, rows_v, sem):
    wid = lax.axis_index("s") * NUM_CORES + lax.axis_index("c")
    base = wid * B_PER_W
    pltpu.sync_copy(weeks_hbm.at[pl.ds(base, B_PER_W)], idx_v)

    def body(i, carry):
        pltpu.async_copy(
            table_hbm.at[idx_v.at[pl.ds(i * CHUNK, CHUNK)]], rows_v, sem
        ).wait()
        pltpu.sync_copy(rows_v, out_hbm.at[pl.ds(base + i * CHUNK, CHUNK)])
        return carry

    lax.fori_loop(0, N_CHUNKS, body, 0)


def kernel(weeks, table):
    out = _emb_lookup(weeks.astype(jnp.int32), table)
    return out[:, None, :]


# speedup vs baseline: 2.4372x; 2.4372x over previous
import functools

import jax
import jax.numpy as jnp
from jax import lax
from jax.experimental import pallas as pl
from jax.experimental.pallas import tpu as pltpu
from jax.experimental.pallas import tpu_sc as plsc

D_MODEL = 1024
N_ROWS = 13
BATCH = 16384
NUM_CORES = 2
NUM_SUBCORES = 16
NUM_WORKERS = NUM_CORES * NUM_SUBCORES
B_PER_W = BATCH // NUM_WORKERS

_mesh = plsc.VectorSubcoreMesh(core_axis_name="c", subcore_axis_name="s")


@functools.partial(
    pl.kernel,
    mesh=_mesh,
    out_type=jax.ShapeDtypeStruct((BATCH, D_MODEL), jnp.float32),
    scratch_types=[
        pltpu.VMEM((B_PER_W,), jnp.int32),
        pltpu.VMEM((N_ROWS, D_MODEL), jnp.float32),
        pltpu.SemaphoreType.DMA,
    ],
)
def _emb_lookup(weeks_hbm, table_hbm, out_hbm, idx_v, table_v, sem):
    wid = lax.axis_index("s") * NUM_CORES + lax.axis_index("c")
    base = wid * B_PER_W
    pltpu.sync_copy(weeks_hbm.at[pl.ds(base, B_PER_W)], idx_v)
    pltpu.sync_copy(table_hbm, table_v)

    def issue(g, carry):
        v = idx_v[pl.ds(g * 16, 16)]
        for j in range(16):
            pltpu.async_copy(
                table_v.at[pl.ds(v[j], 1)],
                out_hbm.at[pl.ds(base + g * 16 + j, 1)],
                sem,
            )
        return carry

    lax.fori_loop(0, B_PER_W // 16, issue, 0)

    def drain(b, carry):
        pltpu.make_async_copy(
            table_v.at[pl.ds(0, 1)], out_hbm.at[pl.ds(base, 1)], sem
        ).wait()
        return carry

    lax.fori_loop(0, B_PER_W, drain, 0)


def kernel(weeks, table):
    out = _emb_lookup(weeks.astype(jnp.int32), table)
    return out[:, None, :]


# speedup vs baseline: 5.2922x; 2.1714x over previous
import functools

import jax
import jax.numpy as jnp
from jax import lax
from jax.experimental import pallas as pl
from jax.experimental.pallas import tpu as pltpu
from jax.experimental.pallas import tpu_sc as plsc

D_MODEL = 1024
N_ROWS = 13
BATCH = 16384
NUM_CORES = 2
NUM_SUBCORES = 16
NUM_WORKERS = NUM_CORES * NUM_SUBCORES
B_PER_W = BATCH // NUM_WORKERS

_mesh = plsc.VectorSubcoreMesh(core_axis_name="c", subcore_axis_name="s")


@functools.partial(
    pl.kernel,
    mesh=_mesh,
    out_type=jax.ShapeDtypeStruct((BATCH, D_MODEL), jnp.float32),
    compiler_params=pltpu.CompilerParams(use_tc_tiling_on_sc=False),
    scratch_types=[
        pltpu.VMEM((B_PER_W,), jnp.int32),
        pltpu.VMEM((N_ROWS, D_MODEL), jnp.float32),
        pltpu.SemaphoreType.DMA,
    ],
)
def _emb_lookup(weeks_hbm, table_hbm, out_hbm, idx_v, table_v, sem):
    wid = lax.axis_index("s") * NUM_CORES + lax.axis_index("c")
    base = wid * B_PER_W
    pltpu.sync_copy(weeks_hbm.at[pl.ds(base, B_PER_W)], idx_v)
    pltpu.sync_copy(table_hbm, table_v)

    def issue(g, carry):
        v = idx_v[pl.ds(g * 16, 16)]
        for j in range(16):
            pltpu.async_copy(
                table_v.at[pl.ds(v[j], 1)],
                out_hbm.at[pl.ds(base + g * 16 + j, 1)],
                sem,
            )
        return carry

    lax.fori_loop(0, B_PER_W // 16, issue, 0)

    def drain(b, carry):
        pltpu.make_async_copy(
            table_v.at[pl.ds(0, 1)], out_hbm.at[pl.ds(base, 1)], sem
        ).wait()
        return carry

    lax.fori_loop(0, B_PER_W, drain, 0)


def kernel(weeks, table):
    out = _emb_lookup(weeks.astype(jnp.int32), table)
    return out[:, None, :]


# speedup vs baseline: 5.6704x; 1.0715x over previous
import functools

import jax
import jax.numpy as jnp
from jax import lax
from jax.experimental import pallas as pl
from jax.experimental.pallas import tpu as pltpu
from jax.experimental.pallas import tpu_sc as plsc

D_MODEL = 1024
N_ROWS = 13
BATCH = 16384
NUM_SUBCORES = 16
GROUPS_C0 = 29
GROUPS_C1 = 35
ROWS_C0 = GROUPS_C0 * 16
ROWS_C1 = GROUPS_C1 * 16
assert (ROWS_C0 + ROWS_C1) * NUM_SUBCORES == BATCH

_mesh = plsc.VectorSubcoreMesh(core_axis_name="c", subcore_axis_name="s")


@functools.partial(
    pl.kernel,
    mesh=_mesh,
    out_type=jax.ShapeDtypeStruct((BATCH, D_MODEL), jnp.float32),
    compiler_params=pltpu.CompilerParams(use_tc_tiling_on_sc=False),
    scratch_types=[
        pltpu.VMEM((ROWS_C1,), jnp.int32),
        pltpu.VMEM((N_ROWS, D_MODEL), jnp.float32),
        pltpu.VMEM((16, D_MODEL), jnp.float32),
        pltpu.SemaphoreType.DMA,
    ],
)
def _emb_lookup(weeks_hbm, table_hbm, out_hbm, idx_v, table_v, drain_v, sem):
    c = lax.axis_index("c")
    s = lax.axis_index("s")
    pltpu.sync_copy(table_hbm, table_v)

    def work(base, n_groups):
        pltpu.sync_copy(
            weeks_hbm.at[pl.ds(base, n_groups * 16)],
            idx_v.at[pl.ds(0, n_groups * 16)],
        )

        def issue(g, carry):
            v = idx_v[pl.ds(g * 16, 16)]
            for j in range(16):
                pltpu.async_copy(
                    table_v.at[pl.ds(v[j], 1)],
                    out_hbm.at[pl.ds(base + g * 16 + j, 1)],
                    sem,
                )
            return carry

        lax.fori_loop(0, n_groups, issue, 0)

        def drain(g, carry):
            pltpu.make_async_copy(
                drain_v, out_hbm.at[pl.ds(base, 16)], sem
            ).wait()
            return carry

        lax.fori_loop(0, n_groups, drain, 0)

    @pl.when(c == 0)
    def _():
        work(s * ROWS_C0, GROUPS_C0)

    @pl.when(c == 1)
    def _():
        work(NUM_SUBCORES * ROWS_C0 + s * ROWS_C1, GROUPS_C1)


def kernel(weeks, table):
    out = _emb_lookup(weeks.astype(jnp.int32), table)
    return out[:, None, :]


# speedup vs baseline: 5.7897x; 1.0210x over previous
import functools

import jax
import jax.numpy as jnp
from jax import lax
from jax.experimental import pallas as pl
from jax.experimental.pallas import tpu as pltpu
from jax.experimental.pallas import tpu_sc as plsc

D_MODEL = 1024
N_ROWS = 13
BATCH = 16384
NUM_SUBCORES = 16
GROUPS_C0 = 30
GROUPS_C1 = 34
ROWS_C0 = GROUPS_C0 * 16
ROWS_C1 = GROUPS_C1 * 16
assert (ROWS_C0 + ROWS_C1) * NUM_SUBCORES == BATCH

_mesh = plsc.VectorSubcoreMesh(core_axis_name="c", subcore_axis_name="s")


@functools.partial(
    pl.kernel,
    mesh=_mesh,
    out_type=jax.ShapeDtypeStruct((BATCH, D_MODEL), jnp.float32),
    compiler_params=pltpu.CompilerParams(use_tc_tiling_on_sc=False),
    scratch_types=[
        pltpu.VMEM((ROWS_C1,), jnp.int32),
        pltpu.VMEM((N_ROWS, D_MODEL), jnp.float32),
        pltpu.VMEM((16, D_MODEL), jnp.float32),
        pltpu.SemaphoreType.DMA,
    ],
)
def _emb_lookup(weeks_hbm, table_hbm, out_hbm, idx_v, table_v, drain_v, sem):
    c = lax.axis_index("c")
    s = lax.axis_index("s")
    is_c0 = c == 0
    base = jnp.where(is_c0, s * ROWS_C0, NUM_SUBCORES * ROWS_C0 + s * ROWS_C1)
    n_groups = jnp.where(is_c0, GROUPS_C0, GROUPS_C1)
    pltpu.sync_copy(table_hbm, table_v)
    pltpu.sync_copy(weeks_hbm.at[pl.ds(base, ROWS_C1)], idx_v)

    def issue(g, carry):
        v = idx_v[pl.ds(g * 16, 16)]
        for j in range(16):
            pltpu.async_copy(
                table_v.at[pl.ds(v[j], 1)],
                out_hbm.at[pl.ds(base + g * 16 + j, 1)],
                sem,
            )
        return carry

    lax.fori_loop(0, n_groups, issue, 0)

    def drain(g, carry):
        pltpu.make_async_copy(drain_v, out_hbm.at[pl.ds(base, 16)], sem).wait()
        return carry

    lax.fori_loop(0, n_groups, drain, 0)


def kernel(weeks, table):
    out = _emb_lookup(weeks.astype(jnp.int32), table)
    return out[:, None, :]
